# baseline (speedup 1.0000x reference)
import functools

import jax
import jax.numpy as jnp
from jax import lax
from jax.experimental import pallas as pl
from jax.experimental.pallas import tpu as pltpu
from jax.experimental.pallas import tpu_sc as plsc

N_ITEMS = 1000000
N_FACTORS = 64
BATCH = 16384

_info = plsc.get_sparse_core_info()
NC, NS, L = _info.num_cores, _info.num_subcores, _info.num_lanes
NW = NC * NS
BW = BATCH // NW
CH = 128
NCH = BW // CH
NG = BW // L


def _body(items0_hbm, items1_hbm, table_hbm, out_hbm,
          idx0_v, idx1_v, rows0_v, rows1_v, out_v, sem0, sem1):
    cid = lax.axis_index("c")
    sid = lax.axis_index("s")
    wid = sid * NC + cid
    base = wid * BW

    for j in range(NCH):
        pltpu.sync_copy(items0_hbm.at[pl.ds(base + j * CH, CH)], idx0_v.at[j])
        pltpu.sync_copy(items1_hbm.at[pl.ds(base + j * CH, CH)], idx1_v.at[j])

    copies = []
    for j in range(NCH):
        copies.append(pltpu.async_copy(
            table_hbm.at[idx0_v.at[j]], rows0_v.at[pl.ds(j * CH, CH)], sem0))
        copies.append(pltpu.async_copy(
            table_hbm.at[idx1_v.at[j]], rows1_v.at[pl.ds(j * CH, CH)], sem1))
    for c in copies:
        c.wait()

    lane = lax.iota(jnp.int32, L)

    def g_body(g, _):
        row16 = g * L + lane
        zero = jnp.zeros((L,), jnp.float32)

        def d_body(dd, acc):
            for j in range(8):
                dvec = jnp.full((L,), dd * 8 + j, jnp.int32)
                a = plsc.load_gather(rows0_v, [row16, dvec])
                b = plsc.load_gather(rows1_v, [row16, dvec])
                acc = acc + a * b
            return acc

        acc = lax.fori_loop(0, N_FACTORS // 8, d_body, zero, unroll=False)
        out16 = 1.0 / (1.0 + jnp.exp(-acc))
        out_v[pl.ds(pl.multiple_of(g * L, L), L)] = out16
        return 0

    lax.fori_loop(0, NG, g_body, 0, unroll=False)

    pltpu.sync_copy(out_v, out_hbm.at[pl.ds(base, BW)])


@jax.jit
def _newmf_sc(items0, items1, table):
    mesh = plsc.VectorSubcoreMesh(core_axis_name="c", subcore_axis_name="s")
    f = functools.partial(
        pl.kernel,
        out_type=jax.ShapeDtypeStruct((BATCH,), jnp.float32),
        mesh=mesh,
        scratch_types=[
            pltpu.VMEM((NCH, CH), jnp.int32),
            pltpu.VMEM((NCH, CH), jnp.int32),
            pltpu.VMEM((BW, N_FACTORS), jnp.float32),
            pltpu.VMEM((BW, N_FACTORS), jnp.float32),
            pltpu.VMEM((BW,), jnp.float32),
            pltpu.SemaphoreType.DMA,
            pltpu.SemaphoreType.DMA,
        ],
        compiler_params=pltpu.CompilerParams(
            use_tc_tiling_on_sc=False,
            needs_layout_passes=False,
        ),
    )(_body)
    return f(items0, items1, table)


def kernel(items, item_factors):
    items0 = items[0].astype(jnp.int32)
    items1 = items[1].astype(jnp.int32)
    return _newmf_sc(items0, items1, item_factors)


# speedup vs baseline: 1.4320x; 1.4320x over previous
import functools

import jax
import jax.numpy as jnp
from jax import lax
from jax.experimental import pallas as pl
from jax.experimental.pallas import tpu as pltpu
from jax.experimental.pallas import tpu_sc as plsc

N_ITEMS = 1000000
N_FACTORS = 64
TILE_H = 8
N_TILES = N_ITEMS // TILE_H
BATCH = 16384

_info = plsc.get_sparse_core_info()
NC, NS, L = _info.num_cores, _info.num_subcores, _info.num_lanes
NW = NC * NS
BW = BATCH // NW
PH = 32
NPH = BW // PH
NG = PH // L


def _body(items0_hbm, items1_hbm, table_hbm, out_hbm,
          raw0_v, raw1_v, rows0_v, rows1_v, out_v,
          sem0, sem1):
    cid = lax.axis_index("c")
    sid = lax.axis_index("s")
    wid = sid * NC + cid
    base = wid * BW

    table_view = table_hbm.reshape(N_TILES, TILE_H, N_FACTORS)

    lane = lax.iota(jnp.int32, L)

    def phase(ph, _):
        pbase = base + ph * PH
        pltpu.sync_copy(items0_hbm.at[pl.ds(pbase, PH)], raw0_v)
        pltpu.sync_copy(items1_hbm.at[pl.ds(pbase, PH)], raw1_v)

        copies = []
        for g in range(NG):
            sl = pl.ds(g * L, L)
            t0v = lax.shift_right_logical(raw0_v[sl], 3)
            t1v = lax.shift_right_logical(raw1_v[sl], 3)
            for l in range(L):
                p = g * L + l
                copies.append(pltpu.async_copy(
                    table_view.at[t0v[l]], rows0_v.at[p], sem0))
                copies.append(pltpu.async_copy(
                    table_view.at[t1v[l]], rows1_v.at[p], sem1))
        for c in copies:
            c.wait()

        for g in range(NG):
            sl = pl.ds(g * L, L)
            p16 = jnp.full((L,), g * L, jnp.int32) + lane
            s0 = lax.bitwise_and(raw0_v[sl], 7)
            s1 = lax.bitwise_and(raw1_v[sl], 7)
            zero = jnp.zeros((L,), jnp.float32)

            def d_body(dd, acc):
                for j in range(8):
                    dvec = jnp.full((L,), dd * 8 + j, jnp.int32)
                    a = plsc.load_gather(rows0_v, [p16, s0, dvec])
                    b = plsc.load_gather(rows1_v, [p16, s1, dvec])
                    acc = acc + a * b
                return acc

            acc = lax.fori_loop(0, N_FACTORS // 8, d_body, zero,
                                unroll=False)
            out16 = 1.0 / (1.0 + jnp.exp(-acc))
            out_v[pl.ds(ph * PH + g * L, L)] = out16
        return 0

    lax.fori_loop(0, NPH, phase, 0, unroll=False)

    pltpu.sync_copy(out_v, out_hbm.at[pl.ds(base, BW)])


@jax.jit
def _newmf_sc(items0, items1, table):
    mesh = plsc.VectorSubcoreMesh(core_axis_name="c", subcore_axis_name="s")
    f = functools.partial(
        pl.kernel,
        out_type=jax.ShapeDtypeStruct((BATCH,), jnp.float32),
        mesh=mesh,
        scratch_types=[
            pltpu.VMEM((PH,), jnp.int32),
            pltpu.VMEM((PH,), jnp.int32),
            pltpu.VMEM((PH, TILE_H, N_FACTORS), jnp.float32),
            pltpu.VMEM((PH, TILE_H, N_FACTORS), jnp.float32),
            pltpu.VMEM((BW,), jnp.float32),
            pltpu.SemaphoreType.DMA,
            pltpu.SemaphoreType.DMA,
        ],
        compiler_params=pltpu.CompilerParams(
            use_tc_tiling_on_sc=True,
            needs_layout_passes=False,
        ),
    )(_body)
    return f(items0, items1, table)


def kernel(items, item_factors):
    items0 = items[0].astype(jnp.int32)
    items1 = items[1].astype(jnp.int32)
    return _newmf_sc(items0, items1, item_factors)
